# baseline (speedup 1.0000x reference)
import jax
import jax.numpy as jnp
from jax import lax
from jax.experimental import pallas as pl
from jax.experimental.pallas import tpu as pltpu
from jax.experimental.pallas import tpu_sc as plsc

_B = 16384
_C = 5
_D = 64
_NW = 32
_BW = _B // _NW
_CH = 128
_NCH = _BW // _CH

_PARAMS = dict(
    compiler_params=pltpu.CompilerParams(use_tc_tiling_on_sc=False,
                                         needs_layout_passes=False),
)


def _bf16_rows_to_f32(src_ref, dst_ref, nrows, iota):
    idx_even = iota * 2
    idx_odd = iota * 2 + 1

    def conv(r, carry):
        rv = iota * 0 + r
        for h in range(2):
            raw = src_ref[r, pl.ds(32 * h, 32)]
            w = plsc.bitcast(raw, jnp.int32)
            lo = plsc.bitcast(lax.shift_left(w, 16), jnp.float32)
            hi = plsc.bitcast(
                lax.bitwise_and(w, jnp.int32(-65536)), jnp.float32)
            plsc.store_scatter(dst_ref, [rv, idx_even + 32 * h], lo)
            plsc.store_scatter(dst_ref, [rv, idx_odd + 32 * h], hi)
        return carry

    lax.fori_loop(0, nrows, conv, 0)


def _k1_body(ctx_idx_hbm, ctx_table, wb_hbm, m_hbm,
             idx_a, ctx_a, idx_b, ctx_b, ctx_f32, m_buf, wb_v,
             sem_a, sem_b):
    wid = lax.axis_index("s") * 2 + lax.axis_index("c")
    base = wid * _BW

    pltpu.sync_copy(wb_hbm, wb_v)
    bufs = [(idx_a, ctx_a, sem_a), (idx_b, ctx_b, sem_b)]

    def start_chunk(ch):
        idx_v, ctx_rows, sem = bufs[ch % 2]
        cb = base + ch * _CH
        pltpu.sync_copy(ctx_idx_hbm.at[cb // _CH], idx_v)
        cps = []
        for j in range(_C):
            cps.append(pltpu.async_copy(
                ctx_table.at[idx_v.at[j]],
                ctx_rows.at[pl.ds(j * _CH, _CH)], sem))
        return cps

    iota = lax.iota(jnp.int32, 16)
    kpats = [iota * 5 + 80 * q for q in range(4)]
    wsc = [plsc.load_gather(wb_v, [iota * 0 + j]) for j in range(_C)]

    pending = start_chunk(0)
    for ch in range(_NCH):
        nxt = start_chunk(ch + 1) if ch + 1 < _NCH else None
        for cp in pending:
            cp.wait()
        _, ctx16, _ = bufs[ch % 2]
        _bf16_rows_to_f32(ctx16, ctx_f32, _C * _CH, iota)
        ctx_rows = ctx_f32

        def body(e, carry):
            r0 = _C * e
            for q in range(4):
                acc = jnp.zeros((16,), jnp.float32)
                for j in range(_C):
                    k = kpats[q] + j
                    rowvec = lax.shift_right_logical(k, 6) + r0
                    colvec = lax.bitwise_and(k, 63)
                    cv = plsc.load_gather(ctx_rows, [rowvec, colvec])
                    acc = acc + cv * wsc[j]
                m_buf[e, pl.ds(16 * q, 16)] = acc
            return carry

        lax.fori_loop(0, _CH, body, 0)
        pltpu.sync_copy(m_buf, m_hbm.at[pl.ds(base + ch * _CH, _CH)])
        pending = nxt


def _k2_body(tgt_idx_hbm, tgt_table, m_hbm, wb_hbm, out_hbm,
             tidx_a, tgt_a, m_a, tidx_b, tgt_b, m_b, tgt_f32, out_v, wb_v,
             sem_a, sem_b):
    wid = lax.axis_index("s") * 2 + lax.axis_index("c")
    base = wid * _BW

    pltpu.sync_copy(wb_hbm, wb_v)
    bufs = [(tidx_a, tgt_a, m_a, sem_a), (tidx_b, tgt_b, m_b, sem_b)]

    def start_chunk(ch):
        tidx_v, tgt_rows, m_rows, sem = bufs[ch % 2]
        cb = base + ch * _CH
        pltpu.sync_copy(tgt_idx_hbm.at[pl.ds(cb, _CH)], tidx_v)
        cps = [pltpu.async_copy(tgt_table.at[tidx_v], tgt_rows, sem),
               pltpu.async_copy(m_hbm.at[pl.ds(cb, _CH)], m_rows, sem)]
        return cps

    iota = lax.iota(jnp.int32, 16)
    bvec = plsc.load_gather(wb_v, [iota * 0 + _C])

    pending = start_chunk(0)
    for ch in range(_NCH):
        nxt = start_chunk(ch + 1) if ch + 1 < _NCH else None
        for cp in pending:
            cp.wait()
        _, tgt16, m_rows, _ = bufs[ch % 2]
        _bf16_rows_to_f32(tgt16, tgt_f32, _CH, iota)
        tgt_rows = tgt_f32

        def gbody(g, carry):
            rowvec = iota + g * 16

            def dbody(dd, acc):
                colvec = iota * 0 + dd
                tv = plsc.load_gather(tgt_rows, [rowvec, colvec])
                mv = plsc.load_gather(m_rows, [rowvec, colvec])
                return acc + tv * mv

            acc = lax.fori_loop(0, _D, dbody, jnp.zeros((16,), jnp.float32))
            x = acc + bvec
            out_v[pl.ds(ch * _CH + g * 16, 16)] = 1.0 / (1.0 + jnp.exp(-x))
            return carry

        lax.fori_loop(0, _CH // 16, gbody, 0)
        pending = nxt

    pltpu.sync_copy(out_v, out_hbm.at[pl.ds(base, _BW)])


@jax.jit
def kernel(context_input, target_input, context_table, target_table,
           W_dense, b_dense):
    ctx_idx = context_input.reshape(_B // _CH, _C, _CH)
    tgt_idx = target_input.reshape(_B)
    ctx16 = context_table.astype(jnp.bfloat16)
    tgt16 = target_table.astype(jnp.bfloat16)
    wb = jnp.concatenate([W_dense.reshape(_C), b_dense,
                          jnp.zeros((2,), jnp.float32)])

    mesh = plsc.VectorSubcoreMesh(core_axis_name="c", subcore_axis_name="s",
                                  num_cores=2, num_subcores=16)
    k1 = pl.kernel(
        _k1_body,
        out_type=jax.ShapeDtypeStruct((_B, _D), jnp.float32),
        mesh=mesh,
        scratch_types=[
            pltpu.VMEM((_C, _CH), jnp.int32),
            pltpu.VMEM((_C * _CH, _D), jnp.bfloat16),
            pltpu.VMEM((_C, _CH), jnp.int32),
            pltpu.VMEM((_C * _CH, _D), jnp.bfloat16),
            pltpu.VMEM((_C * _CH, _D), jnp.float32),
            pltpu.VMEM((_CH, _D), jnp.float32),
            pltpu.VMEM((8,), jnp.float32),
            pltpu.SemaphoreType.DMA,
            pltpu.SemaphoreType.DMA,
        ],
        **_PARAMS,
    )
    m = k1(ctx_idx, ctx16, wb)

    k2 = pl.kernel(
        _k2_body,
        out_type=jax.ShapeDtypeStruct((_B,), jnp.float32),
        mesh=mesh,
        scratch_types=[
            pltpu.VMEM((_CH,), jnp.int32),
            pltpu.VMEM((_CH, _D), jnp.bfloat16),
            pltpu.VMEM((_CH, _D), jnp.float32),
            pltpu.VMEM((_CH,), jnp.int32),
            pltpu.VMEM((_CH, _D), jnp.bfloat16),
            pltpu.VMEM((_CH, _D), jnp.float32),
            pltpu.VMEM((_CH, _D), jnp.float32),
            pltpu.VMEM((_BW,), jnp.float32),
            pltpu.VMEM((8,), jnp.float32),
            pltpu.SemaphoreType.DMA,
            pltpu.SemaphoreType.DMA,
        ],
        **_PARAMS,
    )
    out = k2(tgt_idx, tgt16, m, wb)
    return out.reshape(_B, 1)


# speedup vs baseline: 1.9110x; 1.9110x over previous
import jax
import jax.numpy as jnp
from jax import lax
from jax.experimental import pallas as pl
from jax.experimental.pallas import tpu as pltpu
from jax.experimental.pallas import tpu_sc as plsc

_B = 16384
_C = 5
_D = 64
_V = 1000000
_NW = 32
_BW = _B // _NW
_CHV = 512
_NCHUNK = _V // _CHV + 1
_TSLOT = (_NCHUNK + _NW - 1) // _NW
_LCAP = 16368
_SLOTS = 64
_EB = 64

_PARAMS = dict(
    compiler_params=pltpu.CompilerParams(use_tc_tiling_on_sc=True,
                                         needs_layout_passes=False),
)


def _make_sweep(n_idx):
    nseg = n_idx // 4096
    dump = n_idx

    def body(tT, tailT, idx_hbm, gat_hbm,
             segbuf, lv, lp, chunk, staging, posrow, hv, hp, sem):
        wid = lax.axis_index("s") * 2 + lax.axis_index("c")
        iota = lax.iota(jnp.int32, 16)

        def scan_seg(seg, off0):
            pltpu.sync_copy(idx_hbm.at[pl.ds(seg * 4096, 4096)], segbuf)

            def scan_vec(i, off):
                v = segbuf[pl.ds(16 * i, 16)]
                cid = lax.shift_right_logical(v, 9)
                mine = lax.bitwise_and(cid, 31) == wid
                pos = iota + (seg * 4096 + 16 * i)
                cnt = jnp.squeeze(lax.slice(
                    plsc.all_reduce_population_count(mine), (0,), (1,)))

                @pl.when(off <= _LCAP - 16)
                def _():
                    plsc.store_compressed(lv.at[pl.ds(off, 16)], v, mask=mine)
                    plsc.store_compressed(lp.at[pl.ds(off, 16)], pos, mask=mine)

                return jnp.minimum(off + cnt, _LCAP)

            return lax.fori_loop(0, 256, scan_vec, off0)

        lc = 0
        for seg in range(nseg):
            lc = scan_seg(seg, lc)

        for q in range(_SLOTS // 16):
            posrow[0, pl.ds(16 * q, 16)] = iota * 0 + dump

        ngroups = lax.shift_right_logical(lc + 15, 4)

        def do_chunk(t, slot0):
            cid = wid + _NW * t
            valid = cid < _NCHUNK
            is_tail = cid == _NCHUNK - 1

            @pl.when(valid & jnp.logical_not(is_tail))
            def _():
                off = pl.multiple_of(cid * _CHV, 128)
                pltpu.sync_copy(tT.at[:, pl.ds(off, _CHV)], chunk)

            @pl.when(valid & is_tail)
            def _():
                pltpu.sync_copy(tailT, chunk.at[:, pl.ds(0, 128)])

            base_v = cid * _CHV

            def hscan(g, hoff):
                v = lv[pl.ds(16 * g, 16)]
                p = lp[pl.ds(16 * g, 16)]
                inb = (iota + 16 * g) < lc
                hmask = (lax.shift_right_logical(v, 9) == cid) & inb
                cnt = jnp.squeeze(lax.slice(
                    plsc.all_reduce_population_count(hmask), (0,), (1,)))
                plsc.store_compressed(hv.at[pl.ds(hoff, 16)],
                                      v - base_v, mask=hmask)
                plsc.store_compressed(hp.at[pl.ds(hoff, 16)], p, mask=hmask)
                return hoff + cnt

            nh = lax.fori_loop(0, ngroups, hscan, 0)

            def hproc(i, slot):
                vcol = iota * 0 + jnp.squeeze(
                    lax.slice(hv[pl.ds(i, 16)], (0,), (1,)))
                pos = jnp.squeeze(
                    lax.slice(hp[pl.ds(i, 16)], (0,), (1,)))
                for q in range(4):
                    cv = plsc.load_gather(chunk, [iota + 16 * q, vcol])
                    staging[slot, pl.ds(16 * q, 16)] = cv
                plsc.store_scatter(posrow.at[0], [iota * 0 + slot],
                                   iota * 0 + pos, mask=iota == 0)
                slot = slot + 1

                @pl.when(slot == _SLOTS)
                def _():
                    pltpu.async_copy(staging, gat_hbm.at[posrow.at[0]],
                                     sem).wait()
                    for q in range(_SLOTS // 16):
                        posrow[0, pl.ds(16 * q, 16)] = iota * 0 + dump

                return lax.select(slot == _SLOTS, 0, slot)

            return lax.fori_loop(0, nh, hproc, slot0)

        slot = lax.fori_loop(0, _TSLOT, do_chunk, 0)

        pltpu.async_copy(staging, gat_hbm.at[posrow.at[0]], sem).wait()

    return body


def _kb_body(gctx_hbm, gtgt_hbm, wb_hbm, out_hbm,
             ctx_rows, tgt_rows, out_v, wb_v, sem):
    wid = lax.axis_index("s") * 2 + lax.axis_index("c")
    base = wid * _BW
    iota = lax.iota(jnp.int32, 16)

    pltpu.sync_copy(wb_hbm, wb_v)
    tidx_pats = []
    wpats = []
    for v in range(20):
        k = iota + 16 * v
        tidx_pats.append(k // 5)
        wpats.append(plsc.load_gather(wb_v, [k % 5]))
    bvec = plsc.load_gather(wb_v, [iota * 0 + _C])

    for ch in range(_BW // _EB):
        cb = base + ch * _EB
        cp1 = pltpu.async_copy(gctx_hbm.at[pl.ds(cb * _C, _EB * _C)],
                               ctx_rows, sem)
        cp2 = pltpu.async_copy(gtgt_hbm.at[pl.ds(cb, _EB)], tgt_rows, sem)
        cp1.wait()
        cp2.wait()

        def body(e, carry):
            full_e = iota * 0 + e
            acc = jnp.zeros((16,), jnp.float32)
            for v in range(20):
                c_o, q = divmod(v, 4)
                cvec = ctx_rows[_C * e + c_o, pl.ds(16 * q, 16)]
                texp = plsc.load_gather(tgt_rows, [full_e, tidx_pats[v]])
                acc = acc + cvec * wpats[v] * texp
            s = jnp.zeros((16,), jnp.float32) + jnp.sum(acc)
            plsc.store_scatter(out_v, [full_e + ch * _EB], s,
                               mask=iota == 0)
            return carry

        lax.fori_loop(0, _EB, body, 0)

    for i in range(_BW // 16):
        x = out_v[pl.ds(16 * i, 16)] + bvec
        out_v[pl.ds(16 * i, 16)] = 1.0 / (1.0 + jnp.exp(-x))
    pltpu.sync_copy(out_v, out_hbm.at[pl.ds(base, _BW)])


@jax.jit
def kernel(context_input, target_input, context_table, target_table,
           W_dense, b_dense):
    ctx_idx = context_input.reshape(_B * _C)
    tgt_idx = target_input.reshape(_B)
    wb = jnp.concatenate([W_dense.reshape(_C), b_dense,
                          jnp.zeros((2,), jnp.float32)])
    ctxT = context_table.T
    tgtT = target_table.T
    tail_c = jnp.pad(ctxT[:, _V - _V % _CHV:], ((0, 0), (0, 64)))
    tail_t = jnp.pad(tgtT[:, _V - _V % _CHV:], ((0, 0), (0, 64)))

    mesh = plsc.VectorSubcoreMesh(core_axis_name="c", subcore_axis_name="s",
                                  num_cores=2, num_subcores=16)

    def sweep(n_idx):
        return pl.kernel(
            _make_sweep(n_idx),
            out_type=jax.ShapeDtypeStruct((n_idx + _SLOTS, 128),
                                          jnp.float32),
            mesh=mesh,
            scratch_types=[
                pltpu.VMEM((4096,), jnp.int32),
                pltpu.VMEM((_LCAP,), jnp.int32),
                pltpu.VMEM((_LCAP,), jnp.int32),
                pltpu.VMEM((_D, _CHV), jnp.float32),
                pltpu.VMEM((_SLOTS, 128), jnp.float32),
                pltpu.VMEM((1, _SLOTS), jnp.int32),
                pltpu.VMEM((_LCAP + 16,), jnp.int32),
                pltpu.VMEM((_LCAP + 16,), jnp.int32),
                pltpu.SemaphoreType.DMA,
            ],
            **_PARAMS,
        )

    gctx = sweep(_B * _C)(ctxT, tail_c, ctx_idx)
    gtgt = sweep(_B)(tgtT, tail_t, tgt_idx)

    kb = pl.kernel(
        _kb_body,
        out_type=jax.ShapeDtypeStruct((_B,), jnp.float32),
        mesh=mesh,
        scratch_types=[
            pltpu.VMEM((_EB * _C, 128), jnp.float32),
            pltpu.VMEM((_EB, 128), jnp.float32),
            pltpu.VMEM((_BW,), jnp.float32),
            pltpu.VMEM((8,), jnp.float32),
            pltpu.SemaphoreType.DMA,
        ],
        **_PARAMS,
    )
    out = kb(gctx, gtgt, wb)
    return out.reshape(_B, 1)


# speedup vs baseline: 2.0529x; 1.0743x over previous
import jax
import jax.numpy as jnp
from jax import lax
from jax.experimental import pallas as pl
from jax.experimental.pallas import tpu as pltpu
from jax.experimental.pallas import tpu_sc as plsc

_B = 16384
_C = 5
_D = 64
_V = 1000000
_NW = 32
_BW = _B // _NW
_CHV = 512
_NCHUNK = _V // _CHV + 1
_TSLOT = (_NCHUNK + _NW - 1) // _NW
_LCAP = 16368
_SLOTS = 64
_EB = 64

_PARAMS = dict(
    compiler_params=pltpu.CompilerParams(use_tc_tiling_on_sc=True,
                                         needs_layout_passes=False),
)


def _make_sweep(n_idx):
    nseg = n_idx // 4096
    dump = n_idx

    def body(tT, tailT, idx_hbm, gat_hbm,
             segbuf, lv, lp, chunk, chunk_b, staging, posrow, tmpv, tmpp,
             sem, sem_a, sem_b):
        wid = lax.axis_index("s") * 2 + lax.axis_index("c")
        iota = lax.iota(jnp.int32, 16)

        def scan_seg(seg, off0):
            pltpu.sync_copy(idx_hbm.at[pl.ds(seg * 4096, 4096)], segbuf)

            def scan_vec(i, off):
                v = segbuf[pl.ds(16 * i, 16)]
                cid = lax.shift_right_logical(v, 9)
                mine = lax.bitwise_and(cid, 31) == wid
                pos = iota + (seg * 4096 + 16 * i)
                cnt = jnp.squeeze(lax.slice(
                    plsc.all_reduce_population_count(mine), (0,), (1,)))

                @pl.when(off <= _LCAP - 16)
                def _():
                    plsc.store_compressed(lv.at[pl.ds(off, 16)], v, mask=mine)
                    plsc.store_compressed(lp.at[pl.ds(off, 16)], pos, mask=mine)

                return jnp.minimum(off + cnt, _LCAP)

            return lax.fori_loop(0, 256, scan_vec, off0)

        lc = 0
        for seg in range(nseg):
            lc = scan_seg(seg, lc)

        for q in range(_SLOTS // 16):
            posrow[0, pl.ds(16 * q, 16)] = iota * 0 + dump

        ngroups = lax.shift_right_logical(lc + 15, 4)

        def process(cid, buf, slot0):
            base_v = cid * _CHV

            def hgroup(g, slot):
                v = lv[pl.ds(16 * g, 16)]
                p = lp[pl.ds(16 * g, 16)]
                inb = (iota + 16 * g) < lc
                hmask = (lax.shift_right_logical(v, 9) == cid) & inb
                cnt = jnp.squeeze(lax.slice(
                    plsc.all_reduce_population_count(hmask), (0,), (1,)))
                plsc.store_compressed(tmpv.at[pl.ds(0, 16)],
                                      v - base_v, mask=hmask)
                plsc.store_compressed(tmpp.at[pl.ds(0, 16)], p, mask=hmask)

                def lane(i, slot):
                    vcol = iota * 0 + jnp.squeeze(
                        lax.slice(tmpv[pl.ds(i, 16)], (0,), (1,)))
                    pos = jnp.squeeze(
                        lax.slice(tmpp[pl.ds(i, 16)], (0,), (1,)))
                    for q in range(4):
                        cv = plsc.load_gather(buf, [iota + 16 * q, vcol])
                        staging[slot, pl.ds(16 * q, 16)] = cv
                    plsc.store_scatter(posrow.at[0], [iota * 0 + slot],
                                       iota * 0 + pos, mask=iota == 0)
                    slot = slot + 1

                    @pl.when(slot == _SLOTS)
                    def _():
                        pltpu.async_copy(staging, gat_hbm.at[posrow.at[0]],
                                         sem).wait()
                        for q in range(_SLOTS // 16):
                            posrow[0, pl.ds(16 * q, 16)] = iota * 0 + dump

                    return lax.select(slot == _SLOTS, 0, slot)

                return lax.fori_loop(0, cnt, lane, slot)

            return lax.fori_loop(0, ngroups, hgroup, slot0)

        def start_main(t, buf, dsem):
            cid = wid + _NW * t

            @pl.when(cid < _NCHUNK - 1)
            def _():
                off = pl.multiple_of(cid * _CHV, 128)
                pltpu.async_copy(tT.at[:, pl.ds(off, _CHV)], buf, dsem)

        def wait_main(t, buf, dsem):
            cid = wid + _NW * t

            @pl.when(cid < _NCHUNK - 1)
            def _():
                pltpu.make_async_copy(
                    tT.at[:, pl.ds(0, _CHV)], buf, dsem).wait()

        start_main(0, chunk, sem_a)

        def do_pair(p, slot):
            t0 = 2 * p
            start_main(t0 + 1, chunk_b, sem_b)
            wait_main(t0, chunk, sem_a)
            slot = process(wid + _NW * t0, chunk, slot)
            start_main(t0 + 2, chunk, sem_a)
            wait_main(t0 + 1, chunk_b, sem_b)
            slot = process(wid + _NW * t0 + _NW, chunk_b, slot)
            return slot

        slot = lax.fori_loop(0, _TSLOT // 2, do_pair, 0)

        tail_cid = _NCHUNK - 1

        @pl.when(wid == tail_cid % _NW)
        def _():
            pltpu.sync_copy(tailT, chunk.at[:, pl.ds(0, 128)])

        slot = process(tail_cid, chunk, slot)

        pltpu.async_copy(staging, gat_hbm.at[posrow.at[0]], sem).wait()

    return body


def _kb_body(gctx_hbm, gtgt_hbm, wb_hbm, out_hbm,
             ctx_rows, tgt_rows, out_v, wb_v, sem):
    wid = lax.axis_index("s") * 2 + lax.axis_index("c")
    base = wid * _BW
    iota = lax.iota(jnp.int32, 16)

    pltpu.sync_copy(wb_hbm, wb_v)
    tidx_pats = []
    wpats = []
    for v in range(20):
        k = iota + 16 * v
        tidx_pats.append(k // 5)
        wpats.append(plsc.load_gather(wb_v, [k % 5]))
    bvec = plsc.load_gather(wb_v, [iota * 0 + _C])

    for ch in range(_BW // _EB):
        cb = base + ch * _EB
        cp1 = pltpu.async_copy(gctx_hbm.at[pl.ds(cb * _C, _EB * _C)],
                               ctx_rows, sem)
        cp2 = pltpu.async_copy(gtgt_hbm.at[pl.ds(cb, _EB)], tgt_rows, sem)
        cp1.wait()
        cp2.wait()

        def body(e, carry):
            full_e = iota * 0 + e
            acc = jnp.zeros((16,), jnp.float32)
            for v in range(20):
                c_o, q = divmod(v, 4)
                cvec = ctx_rows[_C * e + c_o, pl.ds(16 * q, 16)]
                texp = plsc.load_gather(tgt_rows, [full_e, tidx_pats[v]])
                acc = acc + cvec * wpats[v] * texp
            s = jnp.zeros((16,), jnp.float32) + jnp.sum(acc)
            plsc.store_scatter(out_v, [full_e + ch * _EB], s,
                               mask=iota == 0)
            return carry

        lax.fori_loop(0, _EB, body, 0)

    for i in range(_BW // 16):
        x = out_v[pl.ds(16 * i, 16)] + bvec
        out_v[pl.ds(16 * i, 16)] = 1.0 / (1.0 + jnp.exp(-x))
    pltpu.sync_copy(out_v, out_hbm.at[pl.ds(base, _BW)])


@jax.jit
def kernel(context_input, target_input, context_table, target_table,
           W_dense, b_dense):
    ctx_idx = context_input.reshape(_B * _C)
    tgt_idx = target_input.reshape(_B)
    wb = jnp.concatenate([W_dense.reshape(_C), b_dense,
                          jnp.zeros((2,), jnp.float32)])
    ctxT = context_table.T
    tgtT = target_table.T
    tail_c = jnp.pad(ctxT[:, _V - _V % _CHV:], ((0, 0), (0, 64)))
    tail_t = jnp.pad(tgtT[:, _V - _V % _CHV:], ((0, 0), (0, 64)))

    mesh = plsc.VectorSubcoreMesh(core_axis_name="c", subcore_axis_name="s",
                                  num_cores=2, num_subcores=16)

    def sweep(n_idx):
        return pl.kernel(
            _make_sweep(n_idx),
            out_type=jax.ShapeDtypeStruct((n_idx + _SLOTS, 128),
                                          jnp.float32),
            mesh=mesh,
            scratch_types=[
                pltpu.VMEM((4096,), jnp.int32),
                pltpu.VMEM((_LCAP,), jnp.int32),
                pltpu.VMEM((_LCAP,), jnp.int32),
                pltpu.VMEM((_D, _CHV), jnp.float32),
                pltpu.VMEM((_D, _CHV), jnp.float32),
                pltpu.VMEM((_SLOTS, 128), jnp.float32),
                pltpu.VMEM((1, _SLOTS), jnp.int32),
                pltpu.VMEM((32,), jnp.int32),
                pltpu.VMEM((32,), jnp.int32),
                pltpu.SemaphoreType.DMA,
                pltpu.SemaphoreType.DMA,
                pltpu.SemaphoreType.DMA,
            ],
            **_PARAMS,
        )

    gctx = sweep(_B * _C)(ctxT, tail_c, ctx_idx)
    gtgt = sweep(_B)(tgtT, tail_t, tgt_idx)

    kb = pl.kernel(
        _kb_body,
        out_type=jax.ShapeDtypeStruct((_B,), jnp.float32),
        mesh=mesh,
        scratch_types=[
            pltpu.VMEM((_EB * _C, 128), jnp.float32),
            pltpu.VMEM((_EB, 128), jnp.float32),
            pltpu.VMEM((_BW,), jnp.float32),
            pltpu.VMEM((8,), jnp.float32),
            pltpu.SemaphoreType.DMA,
        ],
        **_PARAMS,
    )
    out = kb(gctx, gtgt, wb)
    return out.reshape(_B, 1)


# speedup vs baseline: 2.4034x; 1.1707x over previous
import jax
import jax.numpy as jnp
from jax import lax
from jax.experimental import pallas as pl
from jax.experimental.pallas import tpu as pltpu
from jax.experimental.pallas import tpu_sc as plsc

_B = 16384
_C = 5
_D = 64
_V = 1000000
_NW = 32
_BW = _B // _NW
_CHV = 512
_NCHUNK = _V // _CHV + 1
_TSLOT = (_NCHUNK + _NW - 1) // _NW
_BROW = 256
_BCAP = 240
_SLOTS = 64
_EB = 64

_PARAMS = dict(
    compiler_params=pltpu.CompilerParams(use_tc_tiling_on_sc=True,
                                         needs_layout_passes=False),
)


def _make_sweep(n_idx):
    nseg = n_idx // 4096
    dump = n_idx

    def body(tT, tailT, idx_hbm, gat_hbm,
             segbuf, bv, bp, counts, sv, sp, chunk, chunk_b, staging,
             posrow, sem, sem_a, sem_b):
        wid = lax.axis_index("s") * 2 + lax.axis_index("c")
        iota = lax.iota(jnp.int32, 16)
        czero = iota * 0
        lane0 = iota == 0

        for q in range(5):
            counts[pl.ds(16 * q, 16)] = czero

        def scan_seg(seg):
            pltpu.sync_copy(idx_hbm.at[pl.ds(seg * 4096, 4096)], segbuf)

            def scan_vec(i, carry):
                v16 = segbuf[pl.ds(16 * i, 16)]
                cid16 = lax.shift_right_logical(v16, 9)
                mine = lax.bitwise_and(cid16, 31) == wid
                pos16 = iota + (seg * 4096 + 16 * i)
                cnt = jnp.squeeze(lax.slice(
                    plsc.all_reduce_population_count(mine), (0,), (1,)))
                plsc.store_compressed(sv.at[pl.ds(0, 16)], v16, mask=mine)
                plsc.store_compressed(sp.at[pl.ds(0, 16)], pos16, mask=mine)

                def app(j, carry2):
                    vv = jnp.squeeze(lax.slice(
                        sv[pl.ds(j, 16)], (0,), (1,)))
                    pp = jnp.squeeze(lax.slice(
                        sp[pl.ds(j, 16)], (0,), (1,)))
                    t = lax.shift_right_logical(vv, 14)
                    off = jnp.squeeze(lax.slice(
                        counts[pl.ds(t, 16)], (0,), (1,)))
                    okv = lane0 & (off < _BCAP)
                    slotpos = czero + (t * _BROW + off)
                    plsc.store_scatter(
                        bv, [slotpos],
                        czero + lax.bitwise_and(vv, _CHV - 1), mask=okv)
                    plsc.store_scatter(bp, [slotpos], czero + pp, mask=okv)
                    plsc.store_scatter(counts, [czero + t],
                                       czero + (off + 1), mask=okv)
                    return carry2

                lax.fori_loop(0, cnt, app, 0)
                return carry

            lax.fori_loop(0, 256, scan_vec, 0)

        for seg in range(nseg):
            scan_seg(seg)

        dump0 = dump + wid * _SLOTS

        def reset_posrow():
            for q in range(_SLOTS // 16):
                posrow[0, pl.ds(16 * q, 16)] = iota + (dump0 + 16 * q)

        reset_posrow()

        def process(t, buf, slot0, enable):
            tcnt = jnp.squeeze(lax.slice(
                counts[pl.ds(t, 16)], (0,), (1,)))
            tcnt = lax.select(enable, tcnt, 0)

            def lane(j, slot):
                vcol = czero + jnp.squeeze(lax.slice(
                    bv[pl.ds(t * _BROW + j, 16)], (0,), (1,)))
                pos = jnp.squeeze(lax.slice(
                    bp[pl.ds(t * _BROW + j, 16)], (0,), (1,)))
                for q in range(4):
                    cv = plsc.load_gather(buf, [iota + 16 * q, vcol])
                    staging[slot, pl.ds(16 * q, 16)] = cv
                plsc.store_scatter(posrow.at[0], [czero + slot],
                                   czero + pos, mask=lane0)
                slot = slot + 1

                @pl.when(slot == _SLOTS)
                def _():
                    pltpu.async_copy(staging, gat_hbm.at[posrow.at[0]],
                                     sem).wait()
                    reset_posrow()

                return lax.select(slot == _SLOTS, 0, slot)

            return lax.fori_loop(0, tcnt, lane, slot0)

        def start_main(t, buf, dsem):
            cid = wid + _NW * t

            @pl.when(cid < _NCHUNK - 1)
            def _():
                off = pl.multiple_of(cid * _CHV, 128)
                pltpu.async_copy(tT.at[:, pl.ds(off, _CHV)], buf, dsem)

        def wait_main(t, buf, dsem):
            cid = wid + _NW * t

            @pl.when(cid < _NCHUNK - 1)
            def _():
                pltpu.make_async_copy(
                    tT.at[:, pl.ds(0, _CHV)], buf, dsem).wait()

        start_main(0, chunk, sem_a)

        def do_pair(p, slot):
            t0 = 2 * p
            start_main(t0 + 1, chunk_b, sem_b)
            wait_main(t0, chunk, sem_a)
            slot = process(t0, chunk, slot,
                           wid + _NW * t0 < _NCHUNK - 1)
            start_main(t0 + 2, chunk, sem_a)
            wait_main(t0 + 1, chunk_b, sem_b)
            slot = process(t0 + 1, chunk_b, slot,
                           wid + _NW * (t0 + 1) < _NCHUNK - 1)
            return slot

        slot = lax.fori_loop(0, _TSLOT // 2, do_pair, 0)

        tail_cid = _NCHUNK - 1
        is_tail_owner = wid == tail_cid % _NW

        @pl.when(is_tail_owner)
        def _():
            pltpu.sync_copy(tailT, chunk.at[:, pl.ds(0, 128)])

        slot = process(_TSLOT - 1, chunk, slot, is_tail_owner)

        pltpu.async_copy(staging, gat_hbm.at[posrow.at[0]], sem).wait()

    return body


def _kb_body(gctx_hbm, gtgt_hbm, wb_hbm, out_hbm,
             ctx_rows, tgt_rows, out_v, wb_v, sem):
    wid = lax.axis_index("s") * 2 + lax.axis_index("c")
    base = wid * _BW
    iota = lax.iota(jnp.int32, 16)

    pltpu.sync_copy(wb_hbm, wb_v)
    tidx_pats = []
    wpats = []
    for v in range(20):
        k = iota + 16 * v
        tidx_pats.append(k // 5)
        wpats.append(plsc.load_gather(wb_v, [k % 5]))
    bvec = plsc.load_gather(wb_v, [iota * 0 + _C])

    for ch in range(_BW // _EB):
        cb = base + ch * _EB
        cp1 = pltpu.async_copy(gctx_hbm.at[pl.ds(cb * _C, _EB * _C)],
                               ctx_rows, sem)
        cp2 = pltpu.async_copy(gtgt_hbm.at[pl.ds(cb, _EB)], tgt_rows, sem)
        cp1.wait()
        cp2.wait()

        def body(e, carry):
            full_e = iota * 0 + e
            acc = jnp.zeros((16,), jnp.float32)
            for v in range(20):
                c_o, q = divmod(v, 4)
                cvec = ctx_rows[_C * e + c_o, pl.ds(16 * q, 16)]
                texp = plsc.load_gather(tgt_rows, [full_e, tidx_pats[v]])
                acc = acc + cvec * wpats[v] * texp
            s = jnp.zeros((16,), jnp.float32) + jnp.sum(acc)
            plsc.store_scatter(out_v, [full_e + ch * _EB], s,
                               mask=iota == 0)
            return carry

        lax.fori_loop(0, _EB, body, 0)

    for i in range(_BW // 16):
        x = out_v[pl.ds(16 * i, 16)] + bvec
        out_v[pl.ds(16 * i, 16)] = 1.0 / (1.0 + jnp.exp(-x))
    pltpu.sync_copy(out_v, out_hbm.at[pl.ds(base, _BW)])


@jax.jit
def kernel(context_input, target_input, context_table, target_table,
           W_dense, b_dense):
    ctx_idx = context_input.reshape(_B * _C)
    tgt_idx = target_input.reshape(_B)
    wb = jnp.concatenate([W_dense.reshape(_C), b_dense,
                          jnp.zeros((2,), jnp.float32)])
    ctxT = context_table.T
    tgtT = target_table.T
    tail_c = jnp.pad(ctxT[:, _V - _V % _CHV:], ((0, 0), (0, 64)))
    tail_t = jnp.pad(tgtT[:, _V - _V % _CHV:], ((0, 0), (0, 64)))

    mesh = plsc.VectorSubcoreMesh(core_axis_name="c", subcore_axis_name="s",
                                  num_cores=2, num_subcores=16)

    def sweep(n_idx):
        return pl.kernel(
            _make_sweep(n_idx),
            out_type=jax.ShapeDtypeStruct((n_idx + _NW * _SLOTS, 128),
                                          jnp.float32),
            mesh=mesh,
            scratch_types=[
                pltpu.VMEM((4096,), jnp.int32),
                pltpu.VMEM((_TSLOT * _BROW,), jnp.int32),
                pltpu.VMEM((_TSLOT * _BROW,), jnp.int32),
                pltpu.VMEM((80,), jnp.int32),
                pltpu.VMEM((32,), jnp.int32),
                pltpu.VMEM((32,), jnp.int32),
                pltpu.VMEM((_D, _CHV), jnp.float32),
                pltpu.VMEM((_D, _CHV), jnp.float32),
                pltpu.VMEM((_SLOTS, 128), jnp.float32),
                pltpu.VMEM((1, _SLOTS), jnp.int32),
                pltpu.SemaphoreType.DMA,
                pltpu.SemaphoreType.DMA,
                pltpu.SemaphoreType.DMA,
            ],
            **_PARAMS,
        )

    gctx = sweep(_B * _C)(ctxT, tail_c, ctx_idx)
    gtgt = sweep(_B)(tgtT, tail_t, tgt_idx)

    kb = pl.kernel(
        _kb_body,
        out_type=jax.ShapeDtypeStruct((_B,), jnp.float32),
        mesh=mesh,
        scratch_types=[
            pltpu.VMEM((_EB * _C, 128), jnp.float32),
            pltpu.VMEM((_EB, 128), jnp.float32),
            pltpu.VMEM((_BW,), jnp.float32),
            pltpu.VMEM((8,), jnp.float32),
            pltpu.SemaphoreType.DMA,
        ],
        **_PARAMS,
    )
    out = kb(gctx, gtgt, wb)
    return out.reshape(_B, 1)
